# baseline (speedup 1.0000x reference)
import functools

import jax
import jax.numpy as jnp
from jax import lax
from jax.experimental import pallas as pl
from jax.experimental.pallas import tpu as pltpu
from jax.experimental.pallas import tpu_sc as plsc

VOCAB = 100000
DEMBED = 128
BATCH = 4096
SEQ = 50

NC = 2
NS = 16
NW = NC * NS
BPR = 2
NSPLIT = 4
SUBB = BATCH // NSPLIT
ROWS_PER_W = SUBB // NW
NCHUNK = ROWS_PER_W // BPR
NBUF = 4
LANES = 16


def _pos_table():
    ep = jnp.tile(jnp.arange(0, DEMBED, 1, dtype=jnp.float32)[None, :], (SEQ, 1))
    ep = ep.at[:, 1::2].set(ep[:, 0::2])
    ep = 1.0 / (10000.0 ** (ep / DEMBED))
    pos = jnp.tile(jnp.arange(0, SEQ, 1, dtype=jnp.float32)[:, None], (1, DEMBED))
    pos = pos * ep
    pos = pos.at[:, 1::2].set(jnp.cos(pos[:, 1::2]))
    pos = pos.at[:, 0::2].set(jnp.sin(pos[:, 0::2]))
    return pos


def _sc_body(x_hbm, table_hbm, pos_hbm, out_hbm, idx_v, pos_v,
             r0, r1, r2, r3, g0, g1, g2, g3, s0, s1, s2, s3):
    rows = [r0, r1, r2, r3]
    gsem = [g0, g1, g2, g3]
    ssem = [s0, s1, s2, s3]

    cid = lax.axis_index("c")
    sid = lax.axis_index("s")
    wid = sid * NC + cid
    batch_base = wid * ROWS_PER_W

    pltpu.sync_copy(x_hbm.at[pl.ds(batch_base, ROWS_PER_W)], idx_v)
    pltpu.sync_copy(pos_hbm, pos_v)

    def issue_gather(c, b):
        for k in range(BPR):
            pltpu.async_copy(
                table_hbm.at[idx_v.at[c * BPR + k]],
                rows[b].at[k, pl.ds(0, SEQ)],
                gsem[b],
            )

    def wait_gather(b):
        for _ in range(BPR):
            pltpu.make_async_copy(
                table_hbm.at[idx_v.at[0]],
                rows[b].at[0, pl.ds(0, SEQ)],
                gsem[b],
            ).wait()

    def issue_store(c, b):
        pltpu.async_copy(
            rows[b], out_hbm.at[pl.ds(batch_base + c * BPR, BPR)], ssem[b]
        )

    def wait_store(b):
        pltpu.make_async_copy(
            rows[b], out_hbm.at[pl.ds(0, BPR)], ssem[b]
        ).wait()

    def add_pos(b):
        def s_step(s, carry):
            for j in range(DEMBED // LANES):
                sl = pl.ds(j * LANES, LANES)
                p = pos_v[s, sl]
                for k in range(BPR):
                    rows[b][k, s, sl] = rows[b][k, s, sl] + p
            return carry
        lax.fori_loop(0, SEQ, s_step, 0)

    issue_gather(0, 0)
    issue_gather(1, 1)

    issue_gather(2, 2)
    wait_gather(0)
    add_pos(0)
    issue_store(0, 0)

    issue_gather(3, 3)
    wait_gather(1)
    add_pos(1)
    issue_store(1, 1)

    def loop_body(o, carry):
        for bp in range(NBUF):
            j = 2 + o * NBUF + bp
            b = (2 + bp) % NBUF
            rb = (b + 2) % NBUF
            wait_store(rb)
            issue_gather(j + 2, rb)
            wait_gather(b)
            add_pos(b)
            issue_store(j, b)
        return carry

    lax.fori_loop(0, (NCHUNK - 4) // NBUF, loop_body, 0)

    wait_gather(2)
    add_pos(2)
    issue_store(NCHUNK - 2, 2)

    wait_gather(3)
    add_pos(3)
    issue_store(NCHUNK - 1, 3)

    for b in range(NBUF):
        wait_store(b)


@functools.partial(jax.jit, static_argnames=())
def _impl(x, table, pos):
    mesh = plsc.VectorSubcoreMesh(core_axis_name="c", subcore_axis_name="s")
    xi = x.astype(jnp.int32)
    outs = []
    for i in range(NSPLIT):
        outs.append(pl.kernel(
            _sc_body,
            out_type=jax.ShapeDtypeStruct((SUBB, SEQ, DEMBED), jnp.float32),
            mesh=mesh,
            scratch_types=(
                [pltpu.VMEM((ROWS_PER_W, SEQ), jnp.int32),
                 pltpu.VMEM((SEQ, DEMBED), jnp.float32)]
                + [pltpu.VMEM((BPR, SEQ, DEMBED), jnp.float32)] * NBUF
                + [pltpu.SemaphoreType.DMA] * (2 * NBUF)
            ),
        )(lax.slice_in_dim(xi, i * SUBB, (i + 1) * SUBB, axis=0), table, pos))
    return jnp.concatenate(outs, axis=0)


def kernel(x, table):
    return _impl(x, table, _pos_table())


# speedup vs baseline: 1.0445x; 1.0445x over previous
import functools

import jax
import jax.numpy as jnp
from jax import lax
from jax.experimental import pallas as pl
from jax.experimental.pallas import tpu as pltpu
from jax.experimental.pallas import tpu_sc as plsc

VOCAB = 100000
DEMBED = 128
BATCH = 4096
SEQ = 50

NC = 2
NS = 16
NW = NC * NS
BPR = 2
NSPLIT = 4
SUBB = BATCH // NSPLIT
ROWS_PER_W = SUBB // NW
NCHUNK = ROWS_PER_W // BPR
NBUF = 4
LANES = 16


def _pos_table():
    ep = jnp.tile(jnp.arange(0, DEMBED, 1, dtype=jnp.float32)[None, :], (SEQ, 1))
    ep = ep.at[:, 1::2].set(ep[:, 0::2])
    ep = 1.0 / (10000.0 ** (ep / DEMBED))
    pos = jnp.tile(jnp.arange(0, SEQ, 1, dtype=jnp.float32)[:, None], (1, DEMBED))
    pos = pos * ep
    pos = pos.at[:, 1::2].set(jnp.cos(pos[:, 1::2]))
    pos = pos.at[:, 0::2].set(jnp.sin(pos[:, 0::2]))
    return pos


def _sc_body(x_hbm, table_hbm, pos_hbm, out_hbm, idx_v, pos_v,
             r0, r1, r2, r3, g0, g1, g2, g3, s0, s1, s2, s3):
    rows = [r0, r1, r2, r3]
    gsem = [g0, g1, g2, g3]
    ssem = [s0, s1, s2, s3]

    cid = lax.axis_index("c")
    sid = lax.axis_index("s")
    wid = sid * NC + cid
    batch_base = wid * ROWS_PER_W

    pltpu.sync_copy(x_hbm.at[pl.ds(batch_base, ROWS_PER_W)], idx_v)
    pltpu.sync_copy(pos_hbm, pos_v)

    def issue_gather(c, b):
        for k in range(BPR):
            pltpu.async_copy(
                table_hbm.at[idx_v.at[c * BPR + k]],
                rows[b].at[k, pl.ds(0, SEQ)],
                gsem[b],
            )

    def wait_gather(b):
        for _ in range(BPR):
            pltpu.make_async_copy(
                table_hbm.at[idx_v.at[0]],
                rows[b].at[0, pl.ds(0, SEQ)],
                gsem[b],
            ).wait()

    def issue_store(c, b):
        pltpu.async_copy(
            rows[b], out_hbm.at[pl.ds(batch_base + c * BPR, BPR)], ssem[b]
        )

    def wait_store(b):
        pltpu.make_async_copy(
            rows[b], out_hbm.at[pl.ds(0, BPR)], ssem[b]
        ).wait()

    def add_pos(b):
        def s_step(s, carry):
            for j in range(DEMBED // LANES):
                sl = pl.ds(j * LANES, LANES)
                p = pos_v[s, sl]
                for k in range(BPR):
                    rows[b][k, s, sl] = rows[b][k, s, sl] + p
            return carry
        lax.fori_loop(0, SEQ, s_step, 0)

    issue_gather(0, 0)
    issue_gather(1, 1)

    issue_gather(2, 2)
    wait_gather(0)
    add_pos(0)
    issue_store(0, 0)

    issue_gather(3, 3)
    wait_gather(1)
    add_pos(1)
    issue_store(1, 1)

    def loop_body(o, carry):
        for bp in range(NBUF):
            j = 2 + o * NBUF + bp
            b = (2 + bp) % NBUF
            rb = (b + 2) % NBUF
            wait_store(rb)
            issue_gather(j + 2, rb)
            wait_gather(b)
            add_pos(b)
            issue_store(j, b)
        return carry

    lax.fori_loop(0, (NCHUNK - 4) // NBUF, loop_body, 0)

    wait_gather(2)
    add_pos(2)
    issue_store(NCHUNK - 2, 2)

    wait_gather(3)
    add_pos(3)
    issue_store(NCHUNK - 1, 3)

    for b in range(NBUF):
        wait_store(b)


@functools.partial(jax.jit, static_argnames=())
def _impl(x, table, pos):
    mesh = plsc.VectorSubcoreMesh(core_axis_name="c", subcore_axis_name="s")
    xi = x.astype(jnp.int32)
    outs = []
    for i in range(NSPLIT):
        outs.append(pl.kernel(
            _sc_body,
            out_type=jax.ShapeDtypeStruct((SUBB, SEQ, DEMBED), jnp.float32),
            mesh=mesh,
            scratch_types=(
                [pltpu.VMEM((ROWS_PER_W, SEQ), jnp.int32),
                 pltpu.VMEM((SEQ, DEMBED), jnp.float32)]
                + [pltpu.VMEM((BPR, SEQ, DEMBED), jnp.float32)] * NBUF
                + [pltpu.SemaphoreType.DMA] * (2 * NBUF)
            ),
        )(lax.slice_in_dim(xi, i * SUBB, (i + 1) * SUBB, axis=0), table, pos))
    out = jnp.zeros((BATCH, SEQ, DEMBED), jnp.float32)
    for i, o in enumerate(outs):
        out = lax.dynamic_update_slice(out, o, (i * SUBB, 0, 0))
    return out


def kernel(x, table):
    return _impl(x, table, _pos_table())


# speedup vs baseline: 1.8485x; 1.7697x over previous
import functools

import jax
import jax.numpy as jnp
from jax import lax
from jax.experimental import pallas as pl
from jax.experimental.pallas import tpu as pltpu
from jax.experimental.pallas import tpu_sc as plsc

VOCAB = 100000
DEMBED = 128
BATCH = 4096
SEQ = 50

NC = 2
NS = 16
NW = NC * NS
BPR = 2
ROWS_PER_W = BATCH // NW
NCHUNK = ROWS_PER_W // BPR
NBUF = 4
LANES = 16


def _pos_table():
    ep = jnp.tile(jnp.arange(0, DEMBED, 1, dtype=jnp.float32)[None, :], (SEQ, 1))
    ep = ep.at[:, 1::2].set(ep[:, 0::2])
    ep = 1.0 / (10000.0 ** (ep / DEMBED))
    pos = jnp.tile(jnp.arange(0, SEQ, 1, dtype=jnp.float32)[:, None], (1, DEMBED))
    pos = pos * ep
    pos = pos.at[:, 1::2].set(jnp.cos(pos[:, 1::2]))
    pos = pos.at[:, 0::2].set(jnp.sin(pos[:, 0::2]))
    return pos


def _sc_body(x_hbm, table_hbm, pos_hbm, out_hbm, idx_v, pos_v,
             r0, r1, r2, r3, g0, g1, g2, g3, s0, s1, s2, s3):
    rows = [r0, r1, r2, r3]
    gsem = [g0, g1, g2, g3]
    ssem = [s0, s1, s2, s3]

    cid = lax.axis_index("c")
    sid = lax.axis_index("s")
    wid = sid * NC + cid
    batch_base = wid * ROWS_PER_W

    pltpu.sync_copy(x_hbm.at[pl.ds(batch_base, ROWS_PER_W)], idx_v)
    pltpu.sync_copy(pos_hbm, pos_v)

    def issue_gather(c, b):
        for k in range(BPR):
            pltpu.async_copy(
                table_hbm.at[idx_v.at[c * BPR + k]],
                rows[b].at[k, pl.ds(0, SEQ)],
                gsem[b],
            )

    def wait_gather(b):
        for _ in range(BPR):
            pltpu.make_async_copy(
                table_hbm.at[idx_v.at[0]],
                rows[b].at[0, pl.ds(0, SEQ)],
                gsem[b],
            ).wait()

    def issue_store(c, b):
        pltpu.async_copy(
            rows[b], out_hbm.at[pl.ds(batch_base + c * BPR, BPR)], ssem[b]
        )

    def wait_store(b):
        pltpu.make_async_copy(
            rows[b], out_hbm.at[pl.ds(0, BPR)], ssem[b]
        ).wait()

    def add_pos(b):
        def s_step(s, carry):
            for j in range(DEMBED // LANES):
                sl = pl.ds(j * LANES, LANES)
                p = pos_v[s, sl]
                for k in range(BPR):
                    rows[b][k, s, sl] = rows[b][k, s, sl] + p
            return carry
        lax.fori_loop(0, SEQ, s_step, 0)

    issue_gather(0, 0)
    issue_gather(1, 1)

    issue_gather(2, 2)
    wait_gather(0)
    add_pos(0)
    issue_store(0, 0)

    issue_gather(3, 3)
    wait_gather(1)
    add_pos(1)
    issue_store(1, 1)

    def loop_body(o, carry):
        for bp in range(NBUF):
            j = 2 + o * NBUF + bp
            b = (2 + bp) % NBUF
            rb = (b + 2) % NBUF
            wait_store(rb)
            issue_gather(j + 2, rb)
            wait_gather(b)
            add_pos(b)
            issue_store(j, b)
        return carry

    lax.fori_loop(0, (NCHUNK - 4) // NBUF, loop_body, 0)

    wait_gather(2)
    add_pos(2)
    issue_store(NCHUNK - 2, 2)

    wait_gather(3)
    add_pos(3)
    issue_store(NCHUNK - 1, 3)

    for b in range(NBUF):
        wait_store(b)


@functools.partial(jax.jit, static_argnames=())
def _impl(x, table, pos):
    mesh = plsc.VectorSubcoreMesh(core_axis_name="c", subcore_axis_name="s")
    outp = pl.kernel(
        _sc_body,
        out_type=jax.ShapeDtypeStruct((BATCH, SEQ, DEMBED), jnp.float32),
        mesh=mesh,
        scratch_types=(
            [pltpu.VMEM((ROWS_PER_W, SEQ), jnp.int32),
             pltpu.VMEM((SEQ, DEMBED), jnp.float32)]
            + [pltpu.VMEM((BPR, SEQ, DEMBED), jnp.float32)] * NBUF
            + [pltpu.SemaphoreType.DMA] * (2 * NBUF)
        ),
    )(x.astype(jnp.int32), table, pos)
    return outp


def kernel(x, table):
    return _impl(x, table, _pos_table())
